# Initial kernel scaffold; baseline (speedup 1.0000x reference)
import jax
import jax.numpy as jnp
from jax.experimental import pallas as pl


def kernel(x, edge_index, edge_type, node_position, W_rel, W_self, b):
    raise NotImplementedError("write your pallas kernel here")


# speedup vs baseline: 2.7684x; 2.7684x over previous
import functools

import jax
import jax.numpy as jnp
from jax import lax
from jax.experimental import pallas as pl
from jax.experimental.pallas import tpu as pltpu
from jax.experimental.pallas import tpu_sc as plsc

_N = 10000
_E = 160000
_D = 256
_R = 7
_L = 3
_H = 128
_NS = 16
_NC = 2
_EPT = _E // _NS
_K = 128
_NCH = _EPT // _K
_TAIL = _EPT - _NCH * _K
_NPT = 624
_RW = 208


def _mm_body(h_ref, w_ref, b_ref, out_ref):
    r = pl.program_id(1)
    c = pl.program_id(2)
    h0 = h_ref[0]
    h1 = h_ref[1]
    w = w_ref[0]
    acc = jnp.dot(h0, w[:_H, :], preferred_element_type=jnp.float32)
    acc += jnp.dot(h1, w[_H:, :], preferred_element_type=jnp.float32)
    acc += jnp.where(r == _R, 1.0, 0.0) * b_ref[c]
    out_ref[0, 0] = acc


def _tc_matmul(h2, w_all, b2, bn=1000):
    ni = _N // bn
    return pl.pallas_call(
        _mm_body,
        grid=(ni, _R + 1, 2),
        in_specs=[
            pl.BlockSpec((2, bn, _H), lambda i, r, c: (0, i, 0)),
            pl.BlockSpec((1, _D, _H), lambda i, r, c: (r, 0, c)),
            pl.BlockSpec((2, _H), lambda i, r, c: (0, 0)),
        ],
        out_specs=pl.BlockSpec((1, 1, bn, _H), lambda i, r, c: (c, r, i, 0)),
        out_shape=jax.ShapeDtypeStruct((2, _R + 1, _N, _H), jnp.float32),
    )(h2, w_all, b2)


def _sc_body(hw_hbm, src_hbm, et_hbm, dst_hbm, out_hbm,
             src_v, et_v, dst_v, gidx_v, buf_v,
             src_t, et_t, dst_t, gidx_t, buf_t,
             rbuf_v, acc, sem):
    c = lax.axis_index("c")
    s = lax.axis_index("s")
    cbase = c * (8 * _N)

    swb = cbase + _R * _N
    sw0 = swb + s * _NPT
    pltpu.sync_copy(hw_hbm.at[pl.ds(sw0, _NPT)], acc.at[pl.ds(s * _NPT, _NPT)])

    @pl.when(s == _NS - 1)
    def _init_last():
        last = _NS * _NPT
        pltpu.sync_copy(hw_hbm.at[pl.ds(swb + last, _N - last)],
                        acc.at[pl.ds(last, _N - last)])

    plsc.subcore_barrier()

    ebase = s * _EPT

    def chunk(i, carry):
        eb = ebase + i * _K
        pltpu.sync_copy(src_hbm.at[pl.ds(eb, _K)], src_v)
        pltpu.sync_copy(et_hbm.at[pl.ds(eb, _K)], et_v)
        pltpu.sync_copy(dst_hbm.at[pl.ds(eb, _K)], dst_v)
        for j in range(_K // 16):
            sl = pl.ds(j * 16, 16)
            gidx_v[sl] = et_v[sl] * _N + src_v[sl] + cbase
        pltpu.async_copy(hw_hbm.at[gidx_v], buf_v, sem).wait()
        pltpu.sync_copy(buf_v, acc.at[dst_v], add=True)
        return carry

    lax.fori_loop(0, _NCH, chunk, 0)

    eb = ebase + _NCH * _K
    pltpu.sync_copy(src_hbm.at[pl.ds(eb, _TAIL)], src_t)
    pltpu.sync_copy(et_hbm.at[pl.ds(eb, _TAIL)], et_t)
    pltpu.sync_copy(dst_hbm.at[pl.ds(eb, _TAIL)], dst_t)
    gidx_t[...] = et_t[...] * _N + src_t[...] + cbase
    pltpu.async_copy(hw_hbm.at[gidx_t], buf_t, sem).wait()
    pltpu.sync_copy(buf_t, acc.at[dst_t], add=True)

    plsc.subcore_barrier()

    rbase = s * _NPT
    for q in range(_NPT // _RW):
        r0 = rbase + q * _RW
        pltpu.sync_copy(acc.at[pl.ds(r0, _RW)], rbuf_v)

        def relu_row(i, carry):
            for j in range(_H // 16):
                sl = pl.ds(j * 16, 16)
                rbuf_v[i, sl] = jnp.maximum(rbuf_v[i, sl], 0.0)
            return carry

        lax.fori_loop(0, _RW, relu_row, 0)
        pltpu.sync_copy(rbuf_v, out_hbm.at[c, pl.ds(r0, _RW)])

    @pl.when(s == _NS - 1)
    def _write_last():
        last = _NS * _NPT
        pltpu.sync_copy(acc.at[pl.ds(last, _N - last)], buf_t)

        def relu_row_t(i, carry):
            for j in range(_H // 16):
                sl = pl.ds(j * 16, 16)
                buf_t[i, sl] = jnp.maximum(buf_t[i, sl], 0.0)
            return carry

        lax.fori_loop(0, _N - last, relu_row_t, 0)
        pltpu.sync_copy(buf_t, out_hbm.at[c, pl.ds(last, _N - last)])


_sc_edge = functools.partial(
    pl.kernel,
    _sc_body,
    out_type=jax.ShapeDtypeStruct((2, _N, _H), jnp.float32),
    mesh=plsc.VectorSubcoreMesh(core_axis_name="c", subcore_axis_name="s"),
    scratch_types=[
        pltpu.VMEM((_K,), jnp.int32),
        pltpu.VMEM((_K,), jnp.int32),
        pltpu.VMEM((_K,), jnp.int32),
        pltpu.VMEM((_K,), jnp.int32),
        pltpu.VMEM((_K, _H), jnp.float32),
        pltpu.VMEM((_TAIL,), jnp.int32),
        pltpu.VMEM((_TAIL,), jnp.int32),
        pltpu.VMEM((_TAIL,), jnp.int32),
        pltpu.VMEM((_TAIL,), jnp.int32),
        pltpu.VMEM((_TAIL, _H), jnp.float32),
        pltpu.VMEM((_RW, _H), jnp.float32),
        pltpu.VMEM_SHARED((_N, _H), jnp.float32),
        pltpu.SemaphoreType.DMA,
    ],
)()


def kernel(x, edge_index, edge_type, node_position, W_rel, W_self, b):
    src = edge_index[0]
    dst = edge_index[1]
    et = edge_type

    w_all = jnp.concatenate([W_rel, W_self[:, None]], axis=1)

    h2 = x.reshape(_N, 2, _H).transpose(1, 0, 2)
    outs = []
    for l in range(_L):
        hw = _tc_matmul(h2, w_all[l], b[l].reshape(2, _H))
        hw_flat = hw.reshape((2 * (_R + 1)) * _N, _H)
        h2 = _sc_edge(hw_flat, src, et, dst)
        outs.append(h2)

    node_feature = jnp.concatenate(
        [o.transpose(1, 0, 2).reshape(_N, _D) for o in outs], axis=-1)
    return node_feature, node_position
